# baseline (speedup 1.0000x reference)
import functools

import jax
import jax.numpy as jnp
from jax import lax
from jax.experimental import pallas as pl
from jax.experimental.pallas import tpu as pltpu
from jax.experimental.pallas import tpu_sc as plsc

MAXLEN = 200
EMBED = 64
BATCH = 1024
SEQ = 200

NC, NS, LANES = 2, 16, 16
NW = NC * NS
ROWS = BATCH * SEQ
RPW = ROWS // NW
CHUNK = SEQ
NCHUNK = RPW // CHUNK
HALF = CHUNK // 2


def _sc_embed(x2, token_table, pos_table):
    mesh = plsc.VectorSubcoreMesh(core_axis_name="c", subcore_axis_name="s")

    @functools.partial(
        pl.kernel,
        out_type=jax.ShapeDtypeStruct((ROWS, EMBED), jnp.float32),
        mesh=mesh,
        scratch_types=[
            pltpu.VMEM((HALF,), jnp.int32),
            pltpu.VMEM((HALF,), jnp.int32),
            pltpu.VMEM((CHUNK, EMBED), jnp.float32),
            pltpu.VMEM((MAXLEN, EMBED), jnp.float32),
            pltpu.SemaphoreType.DMA,
        ],
        compiler_params=pltpu.CompilerParams(use_tc_tiling_on_sc=False),
    )
    def k(x_hbm, tab_hbm, pos_hbm, out_hbm, idx_a, idx_b, rows_v, pos_v, sem):
        wid = lax.axis_index("s") * NC + lax.axis_index("c")
        pltpu.sync_copy(pos_hbm, pos_v)

        @pl.loop(0, NCHUNK)
        def _(kc):
            g = wid * NCHUNK + kc
            base = g * CHUNK
            pltpu.sync_copy(x_hbm.at[2 * g], idx_a)
            pltpu.sync_copy(x_hbm.at[2 * g + 1], idx_b)
            c1 = pltpu.async_copy(tab_hbm.at[idx_a], rows_v.at[pl.ds(0, HALF)], sem)
            c2 = pltpu.async_copy(tab_hbm.at[idx_b], rows_v.at[pl.ds(HALF, HALF)], sem)
            c1.wait()
            c2.wait()

            @pl.loop(0, CHUNK)
            def _(r):
                for c in range(0, EMBED, LANES):
                    rows_v[r, pl.ds(c, LANES)] = (
                        rows_v[r, pl.ds(c, LANES)] + pos_v[r, pl.ds(c, LANES)]
                    )

            pltpu.sync_copy(rows_v, out_hbm.at[pl.ds(base, CHUNK)])

    return k(x2, token_table, pos_table)


def kernel(x, token_table, pos_table):
    x2 = x.reshape(ROWS // HALF, HALF).astype(jnp.int32)
    out = _sc_embed(x2, token_table, pos_table)
    return out.reshape(BATCH, SEQ, EMBED)
